# baseline (speedup 1.0000x reference)
import jax
import jax.numpy as jnp
from jax import lax
from jax.experimental import pallas as pl
from jax.experimental.pallas import tpu as pltpu
from jax.experimental.pallas import tpu_sc as plsc

N = 10000
E = 320000
DH = 256
HALF = 128
G = 64
NCLS = 3
NC = 2
NS = 16
B = 64
EPT = 20480
BN = 1000


def _fill_f32(ref, n, val):
    def body(i, _):
        ref[pl.ds(i * 16, 16)] = jnp.full((16,), val, jnp.float32)
        return 0
    lax.fori_loop(0, n // 16, body, 0)


def _zero_rows(ref, rows):
    def body(i, _):
        ref[i // 8, pl.ds((i % 8) * 16, 16)] = jnp.zeros((16,), jnp.float32)
        return 0
    lax.fori_loop(0, rows * 8, body, 0)


def _deg_body(dst_hbm, out_hbm, dstb_a, dstb_b, ones, zbuf, acc,
              sem_a, sem_b):
    c = lax.axis_index("c")
    s = lax.axis_index("s")
    DB = 80
    _fill_f32(ones, DB, 1.0)
    _fill_f32(zbuf, 2000, 0.0)

    @pl.when(s < 5)
    def _():
        pltpu.sync_copy(zbuf, acc.at[pl.ds(s * 2000, 2000)])

    plsc.subcore_barrier()

    per_tile = E // NC // NS
    base = c * (E // NC) + s * per_tile

    def start(j, dstb, sem):
        pltpu.async_copy(dst_hbm.at[pl.ds(base + j * DB, DB)], dstb, sem)

    def wait(dstb, sem):
        pltpu.make_async_copy(dst_hbm.at[pl.ds(base, DB)], dstb, sem).wait()

    nchunks = per_tile // DB
    npairs = nchunks // 2
    start(0, dstb_a, sem_a)

    def pair(p, _):
        start(2 * p + 1, dstb_b, sem_b)
        wait(dstb_a, sem_a)
        pltpu.sync_copy(ones, acc.at[dstb_a], add=True)

        @pl.when(p < npairs - 1)
        def _():
            start(2 * p + 2, dstb_a, sem_a)

        wait(dstb_b, sem_b)
        pltpu.sync_copy(ones, acc.at[dstb_b], add=True)
        return 0

    lax.fori_loop(0, npairs, pair, 0)
    start(nchunks - 1, dstb_a, sem_a)
    wait(dstb_a, sem_a)
    pltpu.sync_copy(ones, acc.at[dstb_a], add=True)
    plsc.subcore_barrier()

    @pl.when(s < 5)
    def _():
        pltpu.sync_copy(acc.at[pl.ds(s * 2000, 2000)], zbuf)
        pltpu.sync_copy(zbuf, out_hbm.at[pl.ds(c * N + s * 2000, 2000)])


def _sc_degree(dst):
    mesh = plsc.VectorSubcoreMesh(core_axis_name="c", subcore_axis_name="s")
    k = pl.kernel(
        _deg_body,
        out_type=jax.ShapeDtypeStruct((NC * N,), jnp.float32),
        mesh=mesh,
        scratch_types=[
            pltpu.VMEM((80,), jnp.int32),
            pltpu.VMEM((80,), jnp.int32),
            pltpu.VMEM((80,), jnp.float32),
            pltpu.VMEM((2000,), jnp.float32),
            pltpu.VMEM_SHARED((N,), jnp.float32),
            pltpu.SemaphoreType.DMA,
            pltpu.SemaphoreType.DMA,
        ],
    )
    return k(dst)


def _scat_body(h_hbm, src_hbm, dst_hbm, out_hbm,
               src_all, dst_all, gidx_a, didx_a, gidx_b, didx_b,
               rows_a, rows_b, zrows, acc, sem_ga, sem_gb, sem_sa, sem_sb):
    c = lax.axis_index("c")
    s = lax.axis_index("s")
    _zero_rows(zrows, 16)

    rbase = s * 624
    nch = jnp.where(s == NS - 1, 40, 39)

    def zchunk(j, _):
        pltpu.sync_copy(zrows, acc.at[pl.ds(rbase + j * 16, 16)])
        return 0

    lax.fori_loop(0, nch, zchunk, 0)

    per_tile = EPT
    ebase = s * per_tile
    plsc.subcore_barrier()

    cN = c * N
    EB = 4096
    npairs = EB // B // 2

    def prep(ch, gidx, didx):
        for i in range(B // 16):
            sl = pl.ds(ch * B + i * 16, 16)
            gidx[pl.ds(i * 16, 16)] = src_all[sl] + cN
            didx[pl.ds(i * 16, 16)] = dst_all[sl]

    def startg(gidx, rows, sem):
        pltpu.async_copy(h_hbm.at[gidx], rows, sem)

    def waitg(gidx, rows, sem):
        pltpu.make_async_copy(h_hbm.at[gidx], rows, sem).wait()

    def block(bi, _):
        pltpu.sync_copy(src_hbm.at[pl.ds(ebase + bi * EB, EB)], src_all)
        pltpu.sync_copy(dst_hbm.at[pl.ds(ebase + bi * EB, EB)], dst_all)
        prep(0, gidx_a, didx_a)
        startg(gidx_a, rows_a, sem_ga)

        def pair(p, _):
            prep(2 * p + 1, gidx_b, didx_b)
            startg(gidx_b, rows_b, sem_gb)
            waitg(gidx_a, rows_a, sem_ga)
            pltpu.sync_copy(rows_a, acc.at[didx_a], add=True)

            @pl.when(p < npairs - 1)
            def _():
                prep(2 * p + 2, gidx_a, didx_a)
                startg(gidx_a, rows_a, sem_ga)

            waitg(gidx_b, rows_b, sem_gb)
            pltpu.sync_copy(rows_b, acc.at[didx_b], add=True)
            return 0

        lax.fori_loop(0, npairs, pair, 0)
        return 0

    lax.fori_loop(0, per_tile // EB, block, 0)
    plsc.subcore_barrier()

    def wchunk(j, _):
        r0 = rbase + j * 16
        pltpu.sync_copy(acc.at[pl.ds(r0, 16)], zrows)
        pltpu.sync_copy(zrows, out_hbm.at[pl.ds(cN + r0, 16)])
        return 0

    lax.fori_loop(0, nch, wchunk, 0)


def _sc_scatter(h2, src, dst):
    mesh = plsc.VectorSubcoreMesh(core_axis_name="c", subcore_axis_name="s")
    k = pl.kernel(
        _scat_body,
        out_type=jax.ShapeDtypeStruct((NC * N, HALF), jnp.float32),
        mesh=mesh,
        scratch_types=[
            pltpu.VMEM((4096,), jnp.int32),
            pltpu.VMEM((4096,), jnp.int32),
            pltpu.VMEM((B,), jnp.int32),
            pltpu.VMEM((B,), jnp.int32),
            pltpu.VMEM((B,), jnp.int32),
            pltpu.VMEM((B,), jnp.int32),
            pltpu.VMEM((B, HALF), jnp.float32),
            pltpu.VMEM((B, HALF), jnp.float32),
            pltpu.VMEM((16, HALF), jnp.float32),
            pltpu.VMEM_SHARED((N + 16, HALF), jnp.float32),
            pltpu.SemaphoreType.DMA,
            pltpu.SemaphoreType.DMA,
            pltpu.SemaphoreType.DMA,
            pltpu.SemaphoreType.DMA,
        ],
    )
    return k(h2, src, dst)


def _mm1a_body(x_ref, w_ref, u_ref):
    u_ref[...] = jnp.dot(x_ref[...], w_ref[...],
                         preferred_element_type=jnp.float32,
                         precision=lax.Precision.HIGHEST)


def _tc_mm1a(x, W1):
    return pl.pallas_call(
        _mm1a_body,
        grid=(N // BN,),
        in_specs=[
            pl.BlockSpec((BN, 128), lambda i: (i, 0)),
            pl.BlockSpec((128, DH), lambda i: (0, 0)),
        ],
        out_specs=pl.BlockSpec((BN, DH), lambda i: (i, 0)),
        out_shape=jax.ShapeDtypeStruct((N, DH), jnp.float32),
    )(x, W1)


def _mm1b_body(u_ref, deg_ref, h_ref, inv_ref):
    deg = deg_ref[0] + deg_ref[1] + 1.0
    inv = lax.rsqrt(deg)
    inv_ref[...] = inv
    h_ref[0] = u_ref[...] * inv


def _tc_mm1b(u, deg2):
    grid = (N // BN, NC)
    h, inv = pl.pallas_call(
        _mm1b_body,
        grid=grid,
        in_specs=[
            pl.BlockSpec((BN, HALF), lambda i, c: (i, c)),
            pl.BlockSpec((NC, BN, 1), lambda i, c: (0, i, 0)),
        ],
        out_specs=[
            pl.BlockSpec((1, BN, HALF), lambda i, c: (c, i, 0)),
            pl.BlockSpec((BN, 1), lambda i, c: (i, 0)),
        ],
        out_shape=[
            jax.ShapeDtypeStruct((NC, N, HALF), jnp.float32),
            jax.ShapeDtypeStruct((N, 1), jnp.float32),
        ],
    )(u, deg2)
    return h, inv


def _mm2_body(s_ref, h_ref, inv_ref, b_ref, w_ref, o_ref):
    inv = inv_ref[...]
    t = s_ref[...] + h_ref[...]
    z = jnp.concatenate([t[0], t[1]], axis=1) * inv + b_ref[...]
    z = jnp.maximum(z, 0.0)
    o = jnp.dot(z, w_ref[...], preferred_element_type=jnp.float32,
                precision=lax.Precision.HIGHEST)
    o_ref[0] = o * inv


def _tc_mm23(s2, h2, inv, b, W):
    grid = (N // BN, NC)
    return pl.pallas_call(
        _mm2_body,
        grid=grid,
        in_specs=[
            pl.BlockSpec((NC, BN, HALF), lambda i, c: (0, i, 0)),
            pl.BlockSpec((NC, BN, HALF), lambda i, c: (0, i, 0)),
            pl.BlockSpec((BN, 1), lambda i, c: (i, 0)),
            pl.BlockSpec((1, DH), lambda i, c: (0, 0)),
            pl.BlockSpec((DH, HALF), lambda i, c: (0, c)),
        ],
        out_specs=pl.BlockSpec((1, BN, HALF), lambda i, c: (c, i, 0)),
        out_shape=jax.ShapeDtypeStruct((NC, N, HALF), jnp.float32),
    )(s2, h2, inv, b, W)


def _pool_body(s_ref, h_ref, inv_ref, b_ref, bat_ref, wl_ref, bl_ref,
               o_ref, sums, counts):
    i = pl.program_id(0)

    @pl.when(i == 0)
    def _():
        sums[...] = jnp.zeros_like(sums)
        counts[...] = jnp.zeros_like(counts)

    t = s_ref[...] + h_ref[...]
    h3 = jnp.concatenate([t[0], t[1]], axis=1) * inv_ref[...] + b_ref[...]
    bat = bat_ref[...].reshape(1, BN)
    M = (lax.broadcasted_iota(jnp.int32, (G, BN), 0) == bat)
    Mf = M.astype(jnp.float32)
    sums[...] += jnp.dot(Mf, h3, preferred_element_type=jnp.float32,
                         precision=lax.Precision.HIGHEST)
    counts[...] += jnp.broadcast_to(jnp.sum(Mf, axis=1, keepdims=True),
                                    (G, 128))

    @pl.when(i == N // BN - 1)
    def _():
        pooled = sums[...] / jnp.maximum(counts[:, 0:1], 1.0)
        logits = jnp.dot(pooled, wl_ref[...],
                         preferred_element_type=jnp.float32,
                         precision=lax.Precision.HIGHEST) + bl_ref[...]
        m = jnp.max(logits, axis=1, keepdims=True)
        lse = m + jnp.log(jnp.sum(jnp.exp(logits - m), axis=1, keepdims=True))
        o_ref[...] = logits - lse


def _tc_pool(s3, h3, inv, b3, batch, Wlin, blin):
    grid = (N // BN,)
    return pl.pallas_call(
        _pool_body,
        grid=grid,
        in_specs=[
            pl.BlockSpec((NC, BN, HALF), lambda i: (0, i, 0)),
            pl.BlockSpec((NC, BN, HALF), lambda i: (0, i, 0)),
            pl.BlockSpec((BN, 1), lambda i: (i, 0)),
            pl.BlockSpec((1, DH), lambda i: (0, 0)),
            pl.BlockSpec((BN, 1), lambda i: (i, 0)),
            pl.BlockSpec((DH, NCLS), lambda i: (0, 0)),
            pl.BlockSpec((1, NCLS), lambda i: (0, 0)),
        ],
        out_specs=pl.BlockSpec((G, NCLS), lambda i: (0, 0)),
        out_shape=jax.ShapeDtypeStruct((G, NCLS), jnp.float32),
        scratch_shapes=[
            pltpu.VMEM((G, DH), jnp.float32),
            pltpu.VMEM((G, 128), jnp.float32),
        ],
    )(s3, h3, inv, b3, batch, Wlin, blin)


def kernel(x, edge_index, batch, W1, b1, W2, b2, W3, b3, Wlin, blin):
    src = edge_index[0]
    dst = edge_index[1]

    npad = EPT - E // NS
    srcp = jnp.concatenate(
        [src.reshape(NS, E // NS), jnp.zeros((NS, npad), jnp.int32)],
        axis=1).reshape(-1)
    dstp = jnp.concatenate(
        [dst.reshape(NS, E // NS), jnp.full((NS, npad), N, jnp.int32)],
        axis=1).reshape(-1)

    u1 = _tc_mm1a(x, W1)
    deg2 = _sc_degree(dst).reshape(NC, N, 1)
    h1, inv = _tc_mm1b(u1, deg2)
    s1 = _sc_scatter(h1.reshape(NC * N, HALF), srcp, dstp)
    h2 = _tc_mm23(s1.reshape(NC, N, HALF), h1, inv, b1.reshape(1, DH), W2)
    s2 = _sc_scatter(h2.reshape(NC * N, HALF), srcp, dstp)
    h3 = _tc_mm23(s2.reshape(NC, N, HALF), h2, inv, b2.reshape(1, DH), W3)
    s3 = _sc_scatter(h3.reshape(NC * N, HALF), srcp, dstp)
    return _tc_pool(s3.reshape(NC, N, HALF), h3, inv, b3.reshape(1, DH),
                    batch.reshape(N, 1), Wlin, blin.reshape(1, NCLS))


# speedup vs baseline: 1.0731x; 1.0731x over previous
import jax
import jax.numpy as jnp
from jax import lax
from jax.experimental import pallas as pl
from jax.experimental.pallas import tpu as pltpu
from jax.experimental.pallas import tpu_sc as plsc

N = 10000
E = 320000
DH = 256
HALF = 128
G = 64
NCLS = 3
NC = 2
NS = 16
B = 128
EPT = 20480
BN = 1000


def _fill_f32(ref, n, val):
    def body(i, _):
        ref[pl.ds(i * 16, 16)] = jnp.full((16,), val, jnp.float32)
        return 0
    lax.fori_loop(0, n // 16, body, 0)


def _zero_rows(ref, rows):
    def body(i, _):
        ref[i // 8, pl.ds((i % 8) * 16, 16)] = jnp.zeros((16,), jnp.float32)
        return 0
    lax.fori_loop(0, rows * 8, body, 0)


def _deg_body(dst_hbm, out_hbm, dstb_a, dstb_b, ones, zbuf, acc,
              sem_a, sem_b):
    c = lax.axis_index("c")
    s = lax.axis_index("s")
    DB = 80
    _fill_f32(ones, DB, 1.0)
    _fill_f32(zbuf, 2000, 0.0)

    @pl.when(s < 5)
    def _():
        pltpu.sync_copy(zbuf, acc.at[pl.ds(s * 2000, 2000)])

    plsc.subcore_barrier()

    per_tile = E // NC // NS
    base = c * (E // NC) + s * per_tile

    def start(j, dstb, sem):
        pltpu.async_copy(dst_hbm.at[pl.ds(base + j * DB, DB)], dstb, sem)

    def wait(dstb, sem):
        pltpu.make_async_copy(dst_hbm.at[pl.ds(base, DB)], dstb, sem).wait()

    nchunks = per_tile // DB
    npairs = nchunks // 2
    start(0, dstb_a, sem_a)

    def pair(p, _):
        start(2 * p + 1, dstb_b, sem_b)
        wait(dstb_a, sem_a)
        pltpu.sync_copy(ones, acc.at[dstb_a], add=True)

        @pl.when(p < npairs - 1)
        def _():
            start(2 * p + 2, dstb_a, sem_a)

        wait(dstb_b, sem_b)
        pltpu.sync_copy(ones, acc.at[dstb_b], add=True)
        return 0

    lax.fori_loop(0, npairs, pair, 0)
    start(nchunks - 1, dstb_a, sem_a)
    wait(dstb_a, sem_a)
    pltpu.sync_copy(ones, acc.at[dstb_a], add=True)
    plsc.subcore_barrier()

    @pl.when(s < 5)
    def _():
        pltpu.sync_copy(acc.at[pl.ds(s * 2000, 2000)], zbuf)
        pltpu.sync_copy(zbuf, out_hbm.at[pl.ds(c * N + s * 2000, 2000)])


def _sc_degree(dst):
    mesh = plsc.VectorSubcoreMesh(core_axis_name="c", subcore_axis_name="s")
    k = pl.kernel(
        _deg_body,
        out_type=jax.ShapeDtypeStruct((NC * N,), jnp.float32),
        mesh=mesh,
        scratch_types=[
            pltpu.VMEM((80,), jnp.int32),
            pltpu.VMEM((80,), jnp.int32),
            pltpu.VMEM((80,), jnp.float32),
            pltpu.VMEM((2000,), jnp.float32),
            pltpu.VMEM_SHARED((N,), jnp.float32),
            pltpu.SemaphoreType.DMA,
            pltpu.SemaphoreType.DMA,
        ],
    )
    return k(dst)


def _scat_body(h_hbm, src_hbm, dst_hbm, out_hbm,
               src_all, dst_all, gidx_a, didx_a, gidx_b, didx_b,
               rows_a, rows_b, zrows, acc, sem_ga, sem_gb, sem_sa, sem_sb):
    c = lax.axis_index("c")
    s = lax.axis_index("s")
    _zero_rows(zrows, 16)

    rbase = s * 624
    nch = jnp.where(s == NS - 1, 40, 39)

    def zchunk(j, _):
        pltpu.sync_copy(zrows, acc.at[pl.ds(rbase + j * 16, 16)])
        return 0

    lax.fori_loop(0, nch, zchunk, 0)

    per_tile = EPT
    ebase = s * per_tile
    plsc.subcore_barrier()

    cN = c * N
    EB = 4096
    npairs = EB // B // 2

    def prep(ch, gidx, didx):
        for i in range(B // 16):
            sl = pl.ds(ch * B + i * 16, 16)
            gidx[pl.ds(i * 16, 16)] = src_all[sl] + cN
            didx[pl.ds(i * 16, 16)] = dst_all[sl]

    def startg(gidx, rows, sem):
        pltpu.async_copy(h_hbm.at[gidx], rows, sem)

    def waitg(gidx, rows, sem):
        pltpu.make_async_copy(h_hbm.at[gidx], rows, sem).wait()

    def block(bi, _):
        pltpu.sync_copy(src_hbm.at[pl.ds(ebase + bi * EB, EB)], src_all)
        pltpu.sync_copy(dst_hbm.at[pl.ds(ebase + bi * EB, EB)], dst_all)
        prep(0, gidx_a, didx_a)
        startg(gidx_a, rows_a, sem_ga)

        def pair(p, _):
            prep(2 * p + 1, gidx_b, didx_b)
            startg(gidx_b, rows_b, sem_gb)
            waitg(gidx_a, rows_a, sem_ga)
            pltpu.sync_copy(rows_a, acc.at[didx_a], add=True)

            @pl.when(p < npairs - 1)
            def _():
                prep(2 * p + 2, gidx_a, didx_a)
                startg(gidx_a, rows_a, sem_ga)

            waitg(gidx_b, rows_b, sem_gb)
            pltpu.sync_copy(rows_b, acc.at[didx_b], add=True)
            return 0

        lax.fori_loop(0, npairs, pair, 0)
        return 0

    lax.fori_loop(0, per_tile // EB, block, 0)
    plsc.subcore_barrier()

    def wchunk(j, _):
        r0 = rbase + j * 16
        pltpu.sync_copy(acc.at[pl.ds(r0, 16)], zrows)
        pltpu.sync_copy(zrows, out_hbm.at[pl.ds(cN + r0, 16)])
        return 0

    lax.fori_loop(0, nch, wchunk, 0)


def _sc_scatter(h2, src, dst):
    mesh = plsc.VectorSubcoreMesh(core_axis_name="c", subcore_axis_name="s")
    k = pl.kernel(
        _scat_body,
        out_type=jax.ShapeDtypeStruct((NC * N, HALF), jnp.float32),
        mesh=mesh,
        scratch_types=[
            pltpu.VMEM((4096,), jnp.int32),
            pltpu.VMEM((4096,), jnp.int32),
            pltpu.VMEM((B,), jnp.int32),
            pltpu.VMEM((B,), jnp.int32),
            pltpu.VMEM((B,), jnp.int32),
            pltpu.VMEM((B,), jnp.int32),
            pltpu.VMEM((B, HALF), jnp.float32),
            pltpu.VMEM((B, HALF), jnp.float32),
            pltpu.VMEM((16, HALF), jnp.float32),
            pltpu.VMEM_SHARED((N + 512, HALF), jnp.float32),
            pltpu.SemaphoreType.DMA,
            pltpu.SemaphoreType.DMA,
            pltpu.SemaphoreType.DMA,
            pltpu.SemaphoreType.DMA,
        ],
    )
    return k(h2, src, dst)


def _mm1a_body(x_ref, w_ref, u_ref):
    u_ref[...] = jnp.dot(x_ref[...], w_ref[...],
                         preferred_element_type=jnp.float32,
                         precision=lax.Precision.HIGHEST)


def _tc_mm1a(x, W1):
    return pl.pallas_call(
        _mm1a_body,
        grid=(N // BN,),
        in_specs=[
            pl.BlockSpec((BN, 128), lambda i: (i, 0)),
            pl.BlockSpec((128, DH), lambda i: (0, 0)),
        ],
        out_specs=pl.BlockSpec((BN, DH), lambda i: (i, 0)),
        out_shape=jax.ShapeDtypeStruct((N, DH), jnp.float32),
    )(x, W1)


def _mm1b_body(u_ref, deg_ref, h_ref, inv_ref):
    deg = deg_ref[0] + deg_ref[1] + 1.0
    inv = lax.rsqrt(deg)
    inv_ref[...] = inv
    h_ref[0] = u_ref[...] * inv


def _tc_mm1b(u, deg2):
    grid = (N // BN, NC)
    h, inv = pl.pallas_call(
        _mm1b_body,
        grid=grid,
        in_specs=[
            pl.BlockSpec((BN, HALF), lambda i, c: (i, c)),
            pl.BlockSpec((NC, BN, 1), lambda i, c: (0, i, 0)),
        ],
        out_specs=[
            pl.BlockSpec((1, BN, HALF), lambda i, c: (c, i, 0)),
            pl.BlockSpec((BN, 1), lambda i, c: (i, 0)),
        ],
        out_shape=[
            jax.ShapeDtypeStruct((NC, N, HALF), jnp.float32),
            jax.ShapeDtypeStruct((N, 1), jnp.float32),
        ],
    )(u, deg2)
    return h, inv


def _mm2_body(s_ref, h_ref, inv_ref, b_ref, w_ref, o_ref):
    inv = inv_ref[...]
    t = s_ref[...] + h_ref[...]
    z = jnp.concatenate([t[0], t[1]], axis=1) * inv + b_ref[...]
    z = jnp.maximum(z, 0.0)
    o = jnp.dot(z, w_ref[...], preferred_element_type=jnp.float32,
                precision=lax.Precision.HIGHEST)
    o_ref[0] = o * inv


def _tc_mm23(s2, h2, inv, b, W):
    grid = (N // BN, NC)
    return pl.pallas_call(
        _mm2_body,
        grid=grid,
        in_specs=[
            pl.BlockSpec((NC, BN, HALF), lambda i, c: (0, i, 0)),
            pl.BlockSpec((NC, BN, HALF), lambda i, c: (0, i, 0)),
            pl.BlockSpec((BN, 1), lambda i, c: (i, 0)),
            pl.BlockSpec((1, DH), lambda i, c: (0, 0)),
            pl.BlockSpec((DH, HALF), lambda i, c: (0, c)),
        ],
        out_specs=pl.BlockSpec((1, BN, HALF), lambda i, c: (c, i, 0)),
        out_shape=jax.ShapeDtypeStruct((NC, N, HALF), jnp.float32),
    )(s2, h2, inv, b, W)


def _pool_body(s_ref, h_ref, inv_ref, b_ref, bat_ref, wl_ref, bl_ref,
               o_ref, sums, counts):
    i = pl.program_id(0)

    @pl.when(i == 0)
    def _():
        sums[...] = jnp.zeros_like(sums)
        counts[...] = jnp.zeros_like(counts)

    t = s_ref[...] + h_ref[...]
    h3 = jnp.concatenate([t[0], t[1]], axis=1) * inv_ref[...] + b_ref[...]
    bat = bat_ref[...].reshape(1, BN)
    M = (lax.broadcasted_iota(jnp.int32, (G, BN), 0) == bat)
    Mf = M.astype(jnp.float32)
    sums[...] += jnp.dot(Mf, h3, preferred_element_type=jnp.float32,
                         precision=lax.Precision.HIGHEST)
    counts[...] += jnp.broadcast_to(jnp.sum(Mf, axis=1, keepdims=True),
                                    (G, 128))

    @pl.when(i == N // BN - 1)
    def _():
        pooled = sums[...] / jnp.maximum(counts[:, 0:1], 1.0)
        logits = jnp.dot(pooled, wl_ref[...],
                         preferred_element_type=jnp.float32,
                         precision=lax.Precision.HIGHEST) + bl_ref[...]
        m = jnp.max(logits, axis=1, keepdims=True)
        lse = m + jnp.log(jnp.sum(jnp.exp(logits - m), axis=1, keepdims=True))
        o_ref[...] = logits - lse


def _tc_pool(s3, h3, inv, b3, batch, Wlin, blin):
    grid = (N // BN,)
    return pl.pallas_call(
        _pool_body,
        grid=grid,
        in_specs=[
            pl.BlockSpec((NC, BN, HALF), lambda i: (0, i, 0)),
            pl.BlockSpec((NC, BN, HALF), lambda i: (0, i, 0)),
            pl.BlockSpec((BN, 1), lambda i: (i, 0)),
            pl.BlockSpec((1, DH), lambda i: (0, 0)),
            pl.BlockSpec((BN, 1), lambda i: (i, 0)),
            pl.BlockSpec((DH, NCLS), lambda i: (0, 0)),
            pl.BlockSpec((1, NCLS), lambda i: (0, 0)),
        ],
        out_specs=pl.BlockSpec((G, NCLS), lambda i: (0, 0)),
        out_shape=jax.ShapeDtypeStruct((G, NCLS), jnp.float32),
        scratch_shapes=[
            pltpu.VMEM((G, DH), jnp.float32),
            pltpu.VMEM((G, 128), jnp.float32),
        ],
    )(s3, h3, inv, b3, batch, Wlin, blin)


def kernel(x, edge_index, batch, W1, b1, W2, b2, W3, b3, Wlin, blin):
    src = edge_index[0]
    dst = edge_index[1]

    npad = EPT - E // NS
    pad_dst = N + (jnp.arange(npad, dtype=jnp.int32) % 512)
    srcp = jnp.concatenate(
        [src.reshape(NS, E // NS), jnp.zeros((NS, npad), jnp.int32)],
        axis=1).reshape(-1)
    dstp = jnp.concatenate(
        [dst.reshape(NS, E // NS),
         jnp.broadcast_to(pad_dst, (NS, npad))],
        axis=1).reshape(-1)

    u1 = _tc_mm1a(x, W1)
    deg2 = _sc_degree(dst).reshape(NC, N, 1)
    h1, inv = _tc_mm1b(u1, deg2)
    s1 = _sc_scatter(h1.reshape(NC * N, HALF), srcp, dstp)
    h2 = _tc_mm23(s1.reshape(NC, N, HALF), h1, inv, b1.reshape(1, DH), W2)
    s2 = _sc_scatter(h2.reshape(NC * N, HALF), srcp, dstp)
    h3 = _tc_mm23(s2.reshape(NC, N, HALF), h2, inv, b2.reshape(1, DH), W3)
    s3 = _sc_scatter(h3.reshape(NC * N, HALF), srcp, dstp)
    return _tc_pool(s3.reshape(NC, N, HALF), h3, inv, b3.reshape(1, DH),
                    batch.reshape(N, 1), Wlin, blin.reshape(1, NCLS))


# speedup vs baseline: 2.0150x; 1.8776x over previous
import jax
import jax.numpy as jnp
from jax import lax
from jax.experimental import pallas as pl
from jax.experimental.pallas import tpu as pltpu
from jax.experimental.pallas import tpu_sc as plsc

N = 10000
E = 320000
DH = 256
HALF = 128
G = 64
NCLS = 3
NC = 2
NS = 16
B = 80
BN = 1000


def _fill_f32(ref, n, val):
    def body(i, _):
        ref[pl.ds(i * 16, 16)] = jnp.full((16,), val, jnp.float32)
        return 0
    lax.fori_loop(0, n // 16, body, 0)


def _zero_rows(ref, rows):
    def body(i, _):
        ref[i // 8, pl.ds((i % 8) * 16, 16)] = jnp.zeros((16,), jnp.float32)
        return 0
    lax.fori_loop(0, rows * 8, body, 0)


def _deg_body(dst_hbm, out_hbm, dstb_a, dstb_b, ones, zbuf, acc,
              sem_a, sem_b):
    c = lax.axis_index("c")
    s = lax.axis_index("s")
    DB = 80
    _fill_f32(ones, DB, 1.0)
    _fill_f32(zbuf, 2000, 0.0)

    @pl.when(s < 5)
    def _():
        pltpu.sync_copy(zbuf, acc.at[pl.ds(s * 2000, 2000)])

    plsc.subcore_barrier()

    per_tile = E // NC // NS
    base = c * (E // NC) + s * per_tile

    def start(j, dstb, sem):
        pltpu.async_copy(dst_hbm.at[pl.ds(base + j * DB, DB)], dstb, sem)

    def wait(dstb, sem):
        pltpu.make_async_copy(dst_hbm.at[pl.ds(base, DB)], dstb, sem).wait()

    nchunks = per_tile // DB
    npairs = nchunks // 2
    start(0, dstb_a, sem_a)

    def pair(p, _):
        start(2 * p + 1, dstb_b, sem_b)
        wait(dstb_a, sem_a)
        pltpu.sync_copy(ones, acc.at[dstb_a], add=True)

        @pl.when(p < npairs - 1)
        def _():
            start(2 * p + 2, dstb_a, sem_a)

        wait(dstb_b, sem_b)
        pltpu.sync_copy(ones, acc.at[dstb_b], add=True)
        return 0

    lax.fori_loop(0, npairs, pair, 0)
    start(nchunks - 1, dstb_a, sem_a)
    wait(dstb_a, sem_a)
    pltpu.sync_copy(ones, acc.at[dstb_a], add=True)
    plsc.subcore_barrier()

    @pl.when(s < 5)
    def _():
        pltpu.sync_copy(acc.at[pl.ds(s * 2000, 2000)], zbuf)
        pltpu.sync_copy(zbuf, out_hbm.at[pl.ds(c * N + s * 2000, 2000)])


def _sc_degree(dst):
    mesh = plsc.VectorSubcoreMesh(core_axis_name="c", subcore_axis_name="s")
    k = pl.kernel(
        _deg_body,
        out_type=jax.ShapeDtypeStruct((NC * N,), jnp.float32),
        mesh=mesh,
        scratch_types=[
            pltpu.VMEM((80,), jnp.int32),
            pltpu.VMEM((80,), jnp.int32),
            pltpu.VMEM((80,), jnp.float32),
            pltpu.VMEM((2000,), jnp.float32),
            pltpu.VMEM_SHARED((N,), jnp.float32),
            pltpu.SemaphoreType.DMA,
            pltpu.SemaphoreType.DMA,
        ],
    )
    return k(dst)


def _scat_body(h_hbm, src_hbm, dst_hbm, out_hbm,
               src_all, dst_all, gidx_a, didx_a, gidx_b, didx_b,
               rows_a, rows_b, zrows, acc, sem_ga, sem_gb, sem_sa, sem_sb):
    c = lax.axis_index("c")
    s = lax.axis_index("s")
    _zero_rows(zrows, 16)

    rbase = s * 624
    nch = jnp.where(s == NS - 1, 40, 39)

    def zchunk(j, _):
        pltpu.sync_copy(zrows, acc.at[pl.ds(rbase + j * 16, 16)])
        return 0

    lax.fori_loop(0, nch, zchunk, 0)

    per_tile = E // NS
    ebase = s * per_tile
    plsc.subcore_barrier()

    cN = c * N
    EB = 4000
    npairs = EB // B // 2

    def prep(ch, gidx, didx):
        for i in range(B // 16):
            sl = pl.ds(ch * B + i * 16, 16)
            gidx[pl.ds(i * 16, 16)] = src_all[sl] + cN
            didx[pl.ds(i * 16, 16)] = dst_all[sl]

    def startg(gidx, rows, sem):
        pltpu.async_copy(h_hbm.at[gidx], rows, sem)

    def waitg(gidx, rows, sem):
        pltpu.make_async_copy(h_hbm.at[gidx], rows, sem).wait()

    def block(bi, _):
        pltpu.sync_copy(src_hbm.at[pl.ds(ebase + bi * EB, EB)], src_all)
        pltpu.sync_copy(dst_hbm.at[pl.ds(ebase + bi * EB, EB)], dst_all)
        prep(0, gidx_a, didx_a)
        startg(gidx_a, rows_a, sem_ga)

        def pair(p, _):
            prep(2 * p + 1, gidx_b, didx_b)
            startg(gidx_b, rows_b, sem_gb)
            waitg(gidx_a, rows_a, sem_ga)
            pltpu.sync_copy(rows_a, acc.at[didx_a], add=True)

            @pl.when(p < npairs - 1)
            def _():
                prep(2 * p + 2, gidx_a, didx_a)
                startg(gidx_a, rows_a, sem_ga)

            waitg(gidx_b, rows_b, sem_gb)
            pltpu.sync_copy(rows_b, acc.at[didx_b], add=True)
            return 0

        lax.fori_loop(0, npairs, pair, 0)
        return 0

    lax.fori_loop(0, per_tile // EB, block, 0)
    plsc.subcore_barrier()

    def wchunk(j, _):
        r0 = rbase + j * 16
        pltpu.sync_copy(acc.at[pl.ds(r0, 16)], zrows)
        pltpu.sync_copy(zrows, out_hbm.at[pl.ds(cN + r0, 16)])
        return 0

    lax.fori_loop(0, nch, wchunk, 0)


def _sc_scatter(h2, src, dst):
    mesh = plsc.VectorSubcoreMesh(core_axis_name="c", subcore_axis_name="s")
    k = pl.kernel(
        _scat_body,
        out_type=jax.ShapeDtypeStruct((NC * N, HALF), jnp.float32),
        mesh=mesh,
        scratch_types=[
            pltpu.VMEM((4000,), jnp.int32),
            pltpu.VMEM((4000,), jnp.int32),
            pltpu.VMEM((B,), jnp.int32),
            pltpu.VMEM((B,), jnp.int32),
            pltpu.VMEM((B,), jnp.int32),
            pltpu.VMEM((B,), jnp.int32),
            pltpu.VMEM((B, HALF), jnp.float32),
            pltpu.VMEM((B, HALF), jnp.float32),
            pltpu.VMEM((16, HALF), jnp.float32),
            pltpu.VMEM_SHARED((N, HALF), jnp.float32),
            pltpu.SemaphoreType.DMA,
            pltpu.SemaphoreType.DMA,
            pltpu.SemaphoreType.DMA,
            pltpu.SemaphoreType.DMA,
        ],
    )
    return k(h2, src, dst)


def _mm1a_body(x_ref, w_ref, u_ref):
    u_ref[...] = jnp.dot(x_ref[...], w_ref[...],
                         preferred_element_type=jnp.float32,
                         precision=lax.Precision.HIGHEST)


def _tc_mm1a(x, W1):
    return pl.pallas_call(
        _mm1a_body,
        grid=(N // BN,),
        in_specs=[
            pl.BlockSpec((BN, 128), lambda i: (i, 0)),
            pl.BlockSpec((128, DH), lambda i: (0, 0)),
        ],
        out_specs=pl.BlockSpec((BN, DH), lambda i: (i, 0)),
        out_shape=jax.ShapeDtypeStruct((N, DH), jnp.float32),
    )(x, W1)


def _mm1b_body(u_ref, deg_ref, h_ref, inv_ref):
    deg = deg_ref[0] + deg_ref[1] + 1.0
    inv = lax.rsqrt(deg)
    inv_ref[...] = inv
    h_ref[0] = u_ref[...] * inv


def _tc_mm1b(u, deg2):
    grid = (N // BN, NC)
    h, inv = pl.pallas_call(
        _mm1b_body,
        grid=grid,
        in_specs=[
            pl.BlockSpec((BN, HALF), lambda i, c: (i, c)),
            pl.BlockSpec((NC, BN, 1), lambda i, c: (0, i, 0)),
        ],
        out_specs=[
            pl.BlockSpec((1, BN, HALF), lambda i, c: (c, i, 0)),
            pl.BlockSpec((BN, 1), lambda i, c: (i, 0)),
        ],
        out_shape=[
            jax.ShapeDtypeStruct((NC, N, HALF), jnp.float32),
            jax.ShapeDtypeStruct((N, 1), jnp.float32),
        ],
    )(u, deg2)
    return h, inv


def _mm2_body(s_ref, h_ref, inv_ref, b_ref, w_ref, o_ref):
    inv = inv_ref[...]
    t = s_ref[...] + h_ref[...]
    z = jnp.concatenate([t[0], t[1]], axis=1) * inv + b_ref[...]
    z = jnp.maximum(z, 0.0)
    o = jnp.dot(z, w_ref[...], preferred_element_type=jnp.float32,
                precision=lax.Precision.HIGHEST)
    o_ref[0] = o * inv


def _tc_mm23(s2, h2, inv, b, W):
    grid = (N // BN, NC)
    return pl.pallas_call(
        _mm2_body,
        grid=grid,
        in_specs=[
            pl.BlockSpec((NC, BN, HALF), lambda i, c: (0, i, 0)),
            pl.BlockSpec((NC, BN, HALF), lambda i, c: (0, i, 0)),
            pl.BlockSpec((BN, 1), lambda i, c: (i, 0)),
            pl.BlockSpec((1, DH), lambda i, c: (0, 0)),
            pl.BlockSpec((DH, HALF), lambda i, c: (0, c)),
        ],
        out_specs=pl.BlockSpec((1, BN, HALF), lambda i, c: (c, i, 0)),
        out_shape=jax.ShapeDtypeStruct((NC, N, HALF), jnp.float32),
    )(s2, h2, inv, b, W)


def _pool_body(s_ref, h_ref, inv_ref, b_ref, bat_ref, wl_ref, bl_ref,
               o_ref, sums, counts):
    i = pl.program_id(0)

    @pl.when(i == 0)
    def _():
        sums[...] = jnp.zeros_like(sums)
        counts[...] = jnp.zeros_like(counts)

    t = s_ref[...] + h_ref[...]
    h3 = jnp.concatenate([t[0], t[1]], axis=1) * inv_ref[...] + b_ref[...]
    bat = bat_ref[...].reshape(1, BN)
    M = (lax.broadcasted_iota(jnp.int32, (G, BN), 0) == bat)
    Mf = M.astype(jnp.float32)
    sums[...] += jnp.dot(Mf, h3, preferred_element_type=jnp.float32,
                         precision=lax.Precision.HIGHEST)
    counts[...] += jnp.broadcast_to(jnp.sum(Mf, axis=1, keepdims=True),
                                    (G, 128))

    @pl.when(i == N // BN - 1)
    def _():
        pooled = sums[...] / jnp.maximum(counts[:, 0:1], 1.0)
        logits = jnp.dot(pooled, wl_ref[...],
                         preferred_element_type=jnp.float32,
                         precision=lax.Precision.HIGHEST) + bl_ref[...]
        m = jnp.max(logits, axis=1, keepdims=True)
        lse = m + jnp.log(jnp.sum(jnp.exp(logits - m), axis=1, keepdims=True))
        o_ref[...] = logits - lse


def _tc_pool(s3, h3, inv, b3, batch, Wlin, blin):
    grid = (N // BN,)
    return pl.pallas_call(
        _pool_body,
        grid=grid,
        in_specs=[
            pl.BlockSpec((NC, BN, HALF), lambda i: (0, i, 0)),
            pl.BlockSpec((NC, BN, HALF), lambda i: (0, i, 0)),
            pl.BlockSpec((BN, 1), lambda i: (i, 0)),
            pl.BlockSpec((1, DH), lambda i: (0, 0)),
            pl.BlockSpec((BN, 1), lambda i: (i, 0)),
            pl.BlockSpec((DH, NCLS), lambda i: (0, 0)),
            pl.BlockSpec((1, NCLS), lambda i: (0, 0)),
        ],
        out_specs=pl.BlockSpec((G, NCLS), lambda i: (0, 0)),
        out_shape=jax.ShapeDtypeStruct((G, NCLS), jnp.float32),
        scratch_shapes=[
            pltpu.VMEM((G, DH), jnp.float32),
            pltpu.VMEM((G, 128), jnp.float32),
        ],
    )(s3, h3, inv, b3, batch, Wlin, blin)


def kernel(x, edge_index, batch, W1, b1, W2, b2, W3, b3, Wlin, blin):
    src = edge_index[0]
    dst = edge_index[1]

    u1 = _tc_mm1a(x, W1)
    deg2 = _sc_degree(dst).reshape(NC, N, 1)
    h1, inv = _tc_mm1b(u1, deg2)
    s1 = _sc_scatter(h1.reshape(NC * N, HALF), src, dst)
    h2 = _tc_mm23(s1.reshape(NC, N, HALF), h1, inv, b1.reshape(1, DH), W2)
    s2 = _sc_scatter(h2.reshape(NC * N, HALF), src, dst)
    h3 = _tc_mm23(s2.reshape(NC, N, HALF), h2, inv, b2.reshape(1, DH), W3)
    s3 = _sc_scatter(h3.reshape(NC * N, HALF), src, dst)
    return _tc_pool(s3.reshape(NC, N, HALF), h3, inv, b3.reshape(1, DH),
                    batch.reshape(N, 1), Wlin, blin.reshape(1, NCLS))
